# baseline (speedup 1.0000x reference)
import functools

import jax
import jax.numpy as jnp
from jax import lax
from jax.experimental import pallas as pl
from jax.experimental.pallas import tpu as pltpu
from jax.experimental.pallas import tpu_sc as plsc

_N = 5000
_NPAD = 5024
_NBLK = _NPAD // 16
_NPAIR = _NBLK // 2
_NSUB = 16
_KCAP = ((_NBLK + _NSUB - 1) // _NSUB) * 16
_SCORE_THRESH = 0.05
_NMS_THRESH = 0.5
_MAX_DET = 100
_NEG = -1e9


def _sup16(bx1, by1, bx2, by2, ba, x1s, y1s, x2s, y2s, as_):
    ltx = jnp.maximum(bx1, x1s)
    lty = jnp.maximum(by1, y1s)
    rbx = jnp.minimum(bx2, x2s)
    rby = jnp.minimum(by2, y2s)
    iw = jnp.maximum(rbx - ltx, 0.0)
    ih = jnp.maximum(rby - lty, 0.0)
    inter = iw * ih
    union = ba + as_ - inter
    d = (inter + inter) - union
    return (d > union * jnp.float32(2.0 ** -24)).astype(jnp.int32)


def _iou16(bx1, by1, bx2, by2, ba, x1s, y1s, x2s, y2s, as_, clamp):
    ltx = jnp.maximum(bx1, x1s)
    lty = jnp.maximum(by1, y1s)
    rbx = jnp.minimum(bx2, x2s)
    rby = jnp.minimum(by2, y2s)
    iw = jnp.maximum(rbx - ltx, 0.0)
    ih = jnp.maximum(rby - lty, 0.0)
    inter = iw * ih
    union = ba + as_ - inter
    if clamp:
        union = jnp.maximum(union, 1e-9)
    return inter / union


def _nms_body(x1h, y1h, x2h, y2h, sh, oh, outh,
              x1, y1, x2, y2, sv, ov,
              kb1, kb2, kb3, kb4, kba, keep, outbuf, svec, rbuf, sbuf):
    c = lax.axis_index("c")
    w = lax.axis_index("s")

    @pl.when(c == 0)
    def _():
        pltpu.sync_copy(x1h, x1)
        pltpu.sync_copy(y1h, y1)
        pltpu.sync_copy(x2h, x2)
        pltpu.sync_copy(y2h, y2)
        pltpu.sync_copy(sh, sv)
        pltpu.sync_copy(oh, ov)

        iota = lax.iota(jnp.int32, 16)

        zero16 = jnp.zeros((16,), jnp.float32)

        def zb(r, cc):
            ro = r * 16
            kb1[pl.ds(ro, 16)] = zero16
            kb2[pl.ds(ro, 16)] = zero16
            kb3[pl.ds(ro, 16)] = zero16
            kb4[pl.ds(ro, 16)] = zero16
            kba[pl.ds(ro, 16)] = zero16
            return cc

        lax.fori_loop(0, _KCAP, zb, jnp.int32(0))

        def pair_body(p, carry):
            k_w, ktot_g = carry
            t0 = 2 * p
            o = t0 * 16
            idxa = ov[pl.ds(o, 16)]
            idxb = ov[pl.ds(o + 16, 16)]
            ax1 = plsc.load_gather(x1, [idxa])
            ay1 = plsc.load_gather(y1, [idxa])
            ax2 = plsc.load_gather(x2, [idxa])
            ay2 = plsc.load_gather(y2, [idxa])
            asc = plsc.load_gather(sv, [idxa])
            bx1 = plsc.load_gather(x1, [idxb])
            by1 = plsc.load_gather(y1, [idxb])
            bx2 = plsc.load_gather(x2, [idxb])
            by2 = plsc.load_gather(y2, [idxb])
            bsc = plsc.load_gather(sv, [idxb])
            aar = (ax2 - ax1) * (ay2 - ay1)
            bar = (bx2 - bx1) * (by2 - by1)

            def cbody(k, st):
                supa, supb = st
                for half in range(2):
                    ko = k * 32 + half * 16
                    vx1 = kb1[pl.ds(ko, 16)]
                    vy1 = kb2[pl.ds(ko, 16)]
                    vx2 = kb3[pl.ds(ko, 16)]
                    vy2 = kb4[pl.ds(ko, 16)]
                    va = kba[pl.ds(ko, 16)]
                    supa = supa | _sup16(vx1, vy1, vx2, vy2, va,
                                         ax1, ay1, ax2, ay2, aar)
                    supb = supb | _sup16(vx1, vy1, vx2, vy2, va,
                                         bx1, by1, bx2, by2, bar)
                return supa, supb

            zz = jnp.zeros((16,), jnp.int32)
            supa, supb = lax.fori_loop(
                0, (k_w + 1) >> 1, cbody, (zz, zz))

            par = p & 1
            svec[pl.ds(0, 16)] = supa
            svec[pl.ds(16, 16)] = supb
            pltpu.sync_copy(svec, sbuf.at[pl.ds(par * 512 + w * 32, 32)])
            plsc.subcore_barrier()
            pltpu.sync_copy(sbuf.at[pl.ds(par * 512, 512)], rbuf)
            acca = rbuf[pl.ds(0, 16)]
            accb = rbuf[pl.ds(16, 16)]
            for r in range(1, _NSUB):
                acca = acca | rbuf[pl.ds(r * 32, 16)]
                accb = accb | rbuf[pl.ds(r * 32 + 16, 16)]

            def resolve(gx1, gy1, gx2, gy2, gs, ga, acc, extra):
                keep16 = jnp.zeros((16,), jnp.int32)
                keeps = []
                for j in range(16):
                    xj1 = gx1[j]
                    yj1 = gy1[j]
                    xj2 = gx2[j]
                    yj2 = gy2[j]
                    sj = gs[j]
                    aj = ga[j]
                    valid_j = sj > _SCORE_THRESH
                    cross_j = acc[j] > 0
                    iou = _iou16(gx1, gy1, gx2, gy2, ga,
                                 xj1, yj1, xj2, yj2, aj, True)
                    imask = (iou > _NMS_THRESH) & (keep16 > 0) & (iota < j)
                    intra = plsc.all_reduce_population_count(imask)[0] > 0
                    if extra is not None:
                        ex1, ey1, ex2, ey2, ea, ekeep = extra
                        iou2 = _iou16(ex1, ey1, ex2, ey2, ea,
                                      xj1, yj1, xj2, yj2, aj, True)
                        emask = (iou2 > _NMS_THRESH) & (ekeep > 0)
                        intra = intra | (
                            plsc.all_reduce_population_count(emask)[0] > 0)
                    keep_j = valid_j & jnp.logical_not(cross_j | intra)
                    keeps.append(keep_j)
                    keep16 = keep16 | (
                        (iota == j).astype(jnp.int32)
                        * keep_j.astype(jnp.int32))
                return keep16, keeps

            keepa, keepsa = resolve(ax1, ay1, ax2, ay2, asc, aar, acca, None)
            keepb, keepsb = resolve(bx1, by1, bx2, by2, bsc, bar, accb,
                                    (ax1, ay1, ax2, ay2, aar, keepa))

            owna = w == (t0 & (_NSUB - 1))
            ownb = w == ((t0 + 1) & (_NSUB - 1))
            koff = k_w
            for j in range(16):
                kj = keepsa[j]

                @pl.when(owna & kj)
                def _(j=j, koff=koff):
                    ro = koff * 16
                    kb1[pl.ds(ro, 16)] = jnp.full((16,), ax1[j], jnp.float32)
                    kb2[pl.ds(ro, 16)] = jnp.full((16,), ay1[j], jnp.float32)
                    kb3[pl.ds(ro, 16)] = jnp.full((16,), ax2[j], jnp.float32)
                    kb4[pl.ds(ro, 16)] = jnp.full((16,), ay2[j], jnp.float32)
                    kba[pl.ds(ro, 16)] = jnp.full((16,), aar[j], jnp.float32)

                koff = koff + (owna & kj).astype(jnp.int32)
            for j in range(16):
                kj = keepsb[j]

                @pl.when(ownb & kj)
                def _(j=j, koff=koff):
                    ro = koff * 16
                    kb1[pl.ds(ro, 16)] = jnp.full((16,), bx1[j], jnp.float32)
                    kb2[pl.ds(ro, 16)] = jnp.full((16,), by1[j], jnp.float32)
                    kb3[pl.ds(ro, 16)] = jnp.full((16,), bx2[j], jnp.float32)
                    kb4[pl.ds(ro, 16)] = jnp.full((16,), by2[j], jnp.float32)
                    kba[pl.ds(ro, 16)] = jnp.full((16,), bar[j], jnp.float32)

                koff = koff + (ownb & kj).astype(jnp.int32)

            nka = plsc.all_reduce_population_count(keepa > 0)[0]
            nkb = plsc.all_reduce_population_count(keepb > 0)[0]

            @pl.when(w == 0)
            def _():
                keep[pl.ds(o, 16)] = keepa
                keep[pl.ds(o + 16, 16)] = keepb

            return koff, ktot_g + nka + nkb

        _, ktot = lax.fori_loop(
            0, _NPAIR, pair_body, (jnp.int32(0), jnp.int32(0)))

        @pl.when(w == 0)
        def _():
            def sel_body(t, nk):
                o = t * 16
                kvec = keep[pl.ds(o, 16)]
                cum = jnp.cumsum(kvec)
                exc = cum - kvec
                gidx = o + iota
                kb = kvec > 0
                pos = jnp.where(kb, nk + exc, ktot + gidx - nk - exc)
                m = pos < _MAX_DET
                base = pos * 5
                idx16 = ov[pl.ds(o, 16)]
                vx1 = plsc.load_gather(x1, [idx16])
                vy1 = plsc.load_gather(y1, [idx16])
                vx2 = plsc.load_gather(x2, [idx16])
                vy2 = plsc.load_gather(y2, [idx16])
                vs = plsc.load_gather(sv, [idx16])
                so = jnp.where(kb, vs, jnp.float32(_NEG))
                plsc.store_scatter(outbuf, [base], vx1, mask=m)
                plsc.store_scatter(outbuf, [base + 1], vy1, mask=m)
                plsc.store_scatter(outbuf, [base + 2], vx2, mask=m)
                plsc.store_scatter(outbuf, [base + 3], vy2, mask=m)
                plsc.store_scatter(outbuf, [base + 4], so, mask=m)
                return nk + jnp.sum(kvec)

            lax.fori_loop(0, _NBLK, sel_body, jnp.int32(0))
            pltpu.sync_copy(outbuf, outh)


_nms_call = functools.partial(
    pl.kernel,
    out_type=jax.ShapeDtypeStruct((512,), jnp.float32),
    mesh=plsc.VectorSubcoreMesh(core_axis_name="c", subcore_axis_name="s"),
    compiler_params=pltpu.CompilerParams(needs_layout_passes=False),
    scratch_types=[
        pltpu.VMEM((_NPAD,), jnp.float32),
        pltpu.VMEM((_NPAD,), jnp.float32),
        pltpu.VMEM((_NPAD,), jnp.float32),
        pltpu.VMEM((_NPAD,), jnp.float32),
        pltpu.VMEM((_NPAD,), jnp.float32),
        pltpu.VMEM((_NPAD,), jnp.int32),
        pltpu.VMEM((_KCAP * 16,), jnp.float32),
        pltpu.VMEM((_KCAP * 16,), jnp.float32),
        pltpu.VMEM((_KCAP * 16,), jnp.float32),
        pltpu.VMEM((_KCAP * 16,), jnp.float32),
        pltpu.VMEM((_KCAP * 16,), jnp.float32),
        pltpu.VMEM((_NPAD,), jnp.int32),
        pltpu.VMEM((512,), jnp.float32),
        pltpu.VMEM((32,), jnp.int32),
        pltpu.VMEM((512,), jnp.int32),
        pltpu.VMEM_SHARED((1024,), jnp.int32),
    ],
)(_nms_body)


def kernel(boxes, scores):
    order = jnp.argsort(-scores).astype(jnp.int32)
    pad = _NPAD - _N
    orderp = jnp.concatenate(
        [order, jnp.arange(_N, _NPAD, dtype=jnp.int32)])
    bp = jnp.concatenate([boxes, jnp.zeros((pad, 4), jnp.float32)], axis=0)
    sp = jnp.concatenate(
        [scores, jnp.full((pad,), -1.0, jnp.float32)])
    out = _nms_call(bp[:, 0], bp[:, 1], bp[:, 2], bp[:, 3], sp, orderp)
    return out[:_MAX_DET * 5].reshape(_MAX_DET, 5)


# speedup vs baseline: 1.1879x; 1.1879x over previous
import functools

import jax
import jax.numpy as jnp
from jax import lax
from jax.experimental import pallas as pl
from jax.experimental.pallas import tpu as pltpu
from jax.experimental.pallas import tpu_sc as plsc

_N = 5000
_NPAD = 5024
_NBLK = _NPAD // 16
_NPAIR = _NBLK // 2
_NSUB = 16
_KCAP = ((_NBLK + _NSUB - 1) // _NSUB) * 16
_SCORE_THRESH = 0.05
_NMS_THRESH = 0.5
_MAX_DET = 100
_NEG = -1e9


def _iou16(bx1, by1, bx2, by2, ba, x1s, y1s, x2s, y2s, as_, clamp):
    ltx = jnp.maximum(bx1, x1s)
    lty = jnp.maximum(by1, y1s)
    rbx = jnp.minimum(bx2, x2s)
    rby = jnp.minimum(by2, y2s)
    iw = jnp.maximum(rbx - ltx, 0.0)
    ih = jnp.maximum(rby - lty, 0.0)
    inter = iw * ih
    union = ba + as_ - inter
    if clamp:
        union = jnp.maximum(union, 1e-9)
    return inter / union


def _nms_body(x1h, y1h, x2h, y2h, sh, oh, outh,
              x1, y1, x2, y2, sv, ov,
              kb1, kb2, kb3, kb4, kba, keep, outbuf, svec, rbuf, sbuf):
    c = lax.axis_index("c")
    w = lax.axis_index("s")

    @pl.when(c == 0)
    def _():
        pltpu.sync_copy(x1h, x1)
        pltpu.sync_copy(y1h, y1)
        pltpu.sync_copy(x2h, x2)
        pltpu.sync_copy(y2h, y2)
        pltpu.sync_copy(sh, sv)
        pltpu.sync_copy(oh, ov)

        iota = lax.iota(jnp.int32, 16)

        zero16 = jnp.zeros((16,), jnp.float32)

        def zb(r, cc):
            ro = r * 16
            kb1[pl.ds(ro, 16)] = zero16
            kb2[pl.ds(ro, 16)] = zero16
            kb3[pl.ds(ro, 16)] = zero16
            kb4[pl.ds(ro, 16)] = zero16
            kba[pl.ds(ro, 16)] = zero16
            return cc

        lax.fori_loop(0, _KCAP, zb, jnp.int32(0))

        def pair_body(p, carry):
            k_w, ktot_g = carry
            t0 = 2 * p
            o = t0 * 16
            idxa = ov[pl.ds(o, 16)]
            idxb = ov[pl.ds(o + 16, 16)]
            ax1 = plsc.load_gather(x1, [idxa])
            ay1 = plsc.load_gather(y1, [idxa])
            ax2 = plsc.load_gather(x2, [idxa])
            ay2 = plsc.load_gather(y2, [idxa])
            asc = plsc.load_gather(sv, [idxa])
            bx1 = plsc.load_gather(x1, [idxb])
            by1 = plsc.load_gather(y1, [idxb])
            bx2 = plsc.load_gather(x2, [idxb])
            by2 = plsc.load_gather(y2, [idxb])
            bsc = plsc.load_gather(sv, [idxb])
            aar = (ax2 - ax1) * (ay2 - ay1)
            bar = (bx2 - bx1) * (by2 - by1)

            def cbody(k, st):
                supa, supb = st
                for half in range(2):
                    ko = k * 32 + half * 16
                    vx1 = kb1[pl.ds(ko, 16)]
                    vy1 = kb2[pl.ds(ko, 16)]
                    vx2 = kb3[pl.ds(ko, 16)]
                    vy2 = kb4[pl.ds(ko, 16)]
                    va = kba[pl.ds(ko, 16)]
                    ioua = _iou16(vx1, vy1, vx2, vy2, va,
                                  ax1, ay1, ax2, ay2, aar, False)
                    ioub = _iou16(vx1, vy1, vx2, vy2, va,
                                  bx1, by1, bx2, by2, bar, False)
                    supa = jnp.maximum(supa, ioua)
                    supb = jnp.maximum(supb, ioub)
                return supa, supb

            zz = jnp.zeros((16,), jnp.float32)
            supa, supb = lax.fori_loop(
                0, (k_w + 1) >> 1, cbody, (zz, zz))

            par = p & 1
            svec[pl.ds(0, 16)] = supa
            svec[pl.ds(16, 16)] = supb
            pltpu.sync_copy(svec, sbuf.at[pl.ds(par * 512 + w * 32, 32)])
            plsc.subcore_barrier()
            pltpu.sync_copy(sbuf.at[pl.ds(par * 512, 512)], rbuf)
            acca = rbuf[pl.ds(0, 16)]
            accb = rbuf[pl.ds(16, 16)]
            for r in range(1, _NSUB):
                acca = jnp.maximum(acca, rbuf[pl.ds(r * 32, 16)])
                accb = jnp.maximum(accb, rbuf[pl.ds(r * 32 + 16, 16)])

            def resolve(gx1, gy1, gx2, gy2, gs, ga, acc, extra):
                keep16 = jnp.zeros((16,), jnp.int32)
                for j in range(16):
                    xj1 = gx1[j]
                    yj1 = gy1[j]
                    xj2 = gx2[j]
                    yj2 = gy2[j]
                    sj = gs[j]
                    aj = ga[j]
                    valid_j = sj > _SCORE_THRESH
                    cross_j = acc[j] > _NMS_THRESH
                    iou = _iou16(gx1, gy1, gx2, gy2, ga,
                                 xj1, yj1, xj2, yj2, aj, True)
                    imask = (iou > _NMS_THRESH) & (keep16 > 0) & (iota < j)
                    intra = plsc.all_reduce_population_count(imask)[0] > 0
                    if extra is not None:
                        ex1, ey1, ex2, ey2, ea, ekeep = extra
                        iou2 = _iou16(ex1, ey1, ex2, ey2, ea,
                                      xj1, yj1, xj2, yj2, aj, True)
                        emask = (iou2 > _NMS_THRESH) & (ekeep > 0)
                        intra = intra | (
                            plsc.all_reduce_population_count(emask)[0] > 0)
                    keep_j = valid_j & jnp.logical_not(cross_j | intra)
                    keep16 = keep16 | (
                        (iota == j).astype(jnp.int32)
                        * keep_j.astype(jnp.int32))
                return keep16

            keepa = resolve(ax1, ay1, ax2, ay2, asc, aar, acca, None)
            keepb = resolve(bx1, by1, bx2, by2, bsc, bar, accb,
                            (ax1, ay1, ax2, ay2, aar, keepa))

            selw = iota == w
            mina = plsc.all_reduce_population_count(
                (keepa > 0) & selw)[0] > 0
            minb = plsc.all_reduce_population_count(
                (keepb > 0) & selw)[0] > 0

            @pl.when(mina)
            def _():
                ia = plsc.load_gather(
                    ov, [jnp.full((16,), o + w, jnp.int32)])
                r1 = plsc.load_gather(x1, [ia])
                r2 = plsc.load_gather(y1, [ia])
                r3 = plsc.load_gather(x2, [ia])
                r4 = plsc.load_gather(y2, [ia])
                ro = k_w * 16
                kb1[pl.ds(ro, 16)] = r1
                kb2[pl.ds(ro, 16)] = r2
                kb3[pl.ds(ro, 16)] = r3
                kb4[pl.ds(ro, 16)] = r4
                kba[pl.ds(ro, 16)] = (r3 - r1) * (r4 - r2)

            kw2 = k_w + mina.astype(jnp.int32)

            @pl.when(minb)
            def _():
                ib = plsc.load_gather(
                    ov, [jnp.full((16,), o + 16 + w, jnp.int32)])
                r1 = plsc.load_gather(x1, [ib])
                r2 = plsc.load_gather(y1, [ib])
                r3 = plsc.load_gather(x2, [ib])
                r4 = plsc.load_gather(y2, [ib])
                ro = kw2 * 16
                kb1[pl.ds(ro, 16)] = r1
                kb2[pl.ds(ro, 16)] = r2
                kb3[pl.ds(ro, 16)] = r3
                kb4[pl.ds(ro, 16)] = r4
                kba[pl.ds(ro, 16)] = (r3 - r1) * (r4 - r2)

            koff = kw2 + minb.astype(jnp.int32)

            nka = plsc.all_reduce_population_count(keepa > 0)[0]
            nkb = plsc.all_reduce_population_count(keepb > 0)[0]

            @pl.when(w == 0)
            def _():
                keep[pl.ds(o, 16)] = keepa
                keep[pl.ds(o + 16, 16)] = keepb

            return koff, ktot_g + nka + nkb

        _, ktot = lax.fori_loop(
            0, _NPAIR, pair_body, (jnp.int32(0), jnp.int32(0)))

        @pl.when(w == 0)
        def _():
            def sel_body(t, nk):
                o = t * 16
                kvec = keep[pl.ds(o, 16)]
                cum = jnp.cumsum(kvec)
                exc = cum - kvec
                gidx = o + iota
                kb = kvec > 0
                pos = jnp.where(kb, nk + exc, ktot + gidx - nk - exc)
                m = pos < _MAX_DET
                base = pos * 5
                idx16 = ov[pl.ds(o, 16)]
                vx1 = plsc.load_gather(x1, [idx16])
                vy1 = plsc.load_gather(y1, [idx16])
                vx2 = plsc.load_gather(x2, [idx16])
                vy2 = plsc.load_gather(y2, [idx16])
                vs = plsc.load_gather(sv, [idx16])
                so = jnp.where(kb, vs, jnp.float32(_NEG))
                plsc.store_scatter(outbuf, [base], vx1, mask=m)
                plsc.store_scatter(outbuf, [base + 1], vy1, mask=m)
                plsc.store_scatter(outbuf, [base + 2], vx2, mask=m)
                plsc.store_scatter(outbuf, [base + 3], vy2, mask=m)
                plsc.store_scatter(outbuf, [base + 4], so, mask=m)
                return nk + jnp.sum(kvec)

            lax.fori_loop(0, _NBLK, sel_body, jnp.int32(0))
            pltpu.sync_copy(outbuf, outh)


_nms_call = functools.partial(
    pl.kernel,
    out_type=jax.ShapeDtypeStruct((512,), jnp.float32),
    mesh=plsc.VectorSubcoreMesh(core_axis_name="c", subcore_axis_name="s"),
    compiler_params=pltpu.CompilerParams(needs_layout_passes=False),
    scratch_types=[
        pltpu.VMEM((_NPAD,), jnp.float32),
        pltpu.VMEM((_NPAD,), jnp.float32),
        pltpu.VMEM((_NPAD,), jnp.float32),
        pltpu.VMEM((_NPAD,), jnp.float32),
        pltpu.VMEM((_NPAD,), jnp.float32),
        pltpu.VMEM((_NPAD,), jnp.int32),
        pltpu.VMEM((_KCAP * 16,), jnp.float32),
        pltpu.VMEM((_KCAP * 16,), jnp.float32),
        pltpu.VMEM((_KCAP * 16,), jnp.float32),
        pltpu.VMEM((_KCAP * 16,), jnp.float32),
        pltpu.VMEM((_KCAP * 16,), jnp.float32),
        pltpu.VMEM((_NPAD,), jnp.int32),
        pltpu.VMEM((512,), jnp.float32),
        pltpu.VMEM((32,), jnp.float32),
        pltpu.VMEM((512,), jnp.float32),
        pltpu.VMEM_SHARED((1024,), jnp.float32),
    ],
)(_nms_body)


def kernel(boxes, scores):
    order = jnp.argsort(-scores).astype(jnp.int32)
    pad = _NPAD - _N
    orderp = jnp.concatenate(
        [order, jnp.arange(_N, _NPAD, dtype=jnp.int32)])
    bp = jnp.concatenate([boxes, jnp.zeros((pad, 4), jnp.float32)], axis=0)
    sp = jnp.concatenate(
        [scores, jnp.full((pad,), -1.0, jnp.float32)])
    out = _nms_call(bp[:, 0], bp[:, 1], bp[:, 2], bp[:, 3], sp, orderp)
    return out[:_MAX_DET * 5].reshape(_MAX_DET, 5)


# speedup vs baseline: 1.4787x; 1.2447x over previous
import functools

import jax
import jax.numpy as jnp
from jax import lax
from jax.experimental import pallas as pl
from jax.experimental.pallas import tpu as pltpu
from jax.experimental.pallas import tpu_sc as plsc

_N = 5000
_NPAD = 5024
_NBLK = _NPAD // 16
_NPAIR = _NBLK // 2
_NSUB = 16
_KCAP = ((_NBLK + _NSUB - 1) // _NSUB) * 16
_SCORE_THRESH = 0.05
_NMS_THRESH = 0.5
_MAX_DET = 100
_NEG = -1e9


def _iou16(bx1, by1, bx2, by2, ba, x1s, y1s, x2s, y2s, as_, clamp):
    ltx = jnp.maximum(bx1, x1s)
    lty = jnp.maximum(by1, y1s)
    rbx = jnp.minimum(bx2, x2s)
    rby = jnp.minimum(by2, y2s)
    iw = jnp.maximum(rbx - ltx, 0.0)
    ih = jnp.maximum(rby - lty, 0.0)
    inter = iw * ih
    union = ba + as_ - inter
    if clamp:
        union = jnp.maximum(union, 1e-9)
    return inter / union


def _nms_body(x1h, y1h, x2h, y2h, sh, oh, outh,
              x1, y1, x2, y2, sv, ov,
              kb1, kb2, kb3, kb4, kba, keep, outbuf, svec, rbuf, sbuf):
    c = lax.axis_index("c")
    w = lax.axis_index("s")

    @pl.when(c == 0)
    def _():
        pltpu.sync_copy(x1h, x1)
        pltpu.sync_copy(y1h, y1)
        pltpu.sync_copy(x2h, x2)
        pltpu.sync_copy(y2h, y2)
        pltpu.sync_copy(sh, sv)
        pltpu.sync_copy(oh, ov)

        iota = lax.iota(jnp.int32, 16)

        zero16 = jnp.zeros((16,), jnp.float32)

        def zb(r, cc):
            ro = r * 16
            kb1[pl.ds(ro, 16)] = zero16
            kb2[pl.ds(ro, 16)] = zero16
            kb3[pl.ds(ro, 16)] = zero16
            kb4[pl.ds(ro, 16)] = zero16
            kba[pl.ds(ro, 16)] = zero16
            return cc

        lax.fori_loop(0, _KCAP, zb, jnp.int32(0))

        def pair_body(p, carry):
            k_w, ktot_g = carry
            t0 = 2 * p
            o = t0 * 16
            idxa = ov[pl.ds(o, 16)]
            idxb = ov[pl.ds(o + 16, 16)]
            ax1 = plsc.load_gather(x1, [idxa])
            ay1 = plsc.load_gather(y1, [idxa])
            ax2 = plsc.load_gather(x2, [idxa])
            ay2 = plsc.load_gather(y2, [idxa])
            asc = plsc.load_gather(sv, [idxa])
            bx1 = plsc.load_gather(x1, [idxb])
            by1 = plsc.load_gather(y1, [idxb])
            bx2 = plsc.load_gather(x2, [idxb])
            by2 = plsc.load_gather(y2, [idxb])
            bsc = plsc.load_gather(sv, [idxb])
            aar = (ax2 - ax1) * (ay2 - ay1)
            bar = (bx2 - bx1) * (by2 - by1)

            def cbody(k, st):
                supa, supb = st
                for half in range(2):
                    ko = k * 32 + half * 16
                    vx1 = kb1[pl.ds(ko, 16)]
                    vy1 = kb2[pl.ds(ko, 16)]
                    vx2 = kb3[pl.ds(ko, 16)]
                    vy2 = kb4[pl.ds(ko, 16)]
                    va = kba[pl.ds(ko, 16)]
                    ioua = _iou16(vx1, vy1, vx2, vy2, va,
                                  ax1, ay1, ax2, ay2, aar, False)
                    ioub = _iou16(vx1, vy1, vx2, vy2, va,
                                  bx1, by1, bx2, by2, bar, False)
                    supa = jnp.maximum(supa, ioua)
                    supb = jnp.maximum(supb, ioub)
                return supa, supb

            zz = jnp.zeros((16,), jnp.float32)
            supa, supb = lax.fori_loop(
                0, (k_w + 1) >> 1, cbody, (zz, zz))

            par = p & 1
            svec[pl.ds(0, 16)] = supa
            svec[pl.ds(16, 16)] = supb
            pltpu.sync_copy(svec, sbuf.at[pl.ds(par * 512 + w * 32, 32)])
            plsc.subcore_barrier()
            pltpu.sync_copy(sbuf.at[pl.ds(par * 512, 512)], rbuf)
            acca = rbuf[pl.ds(0, 16)]
            accb = rbuf[pl.ds(16, 16)]
            for r in range(1, _NSUB):
                acca = jnp.maximum(acca, rbuf[pl.ds(r * 32, 16)])
                accb = jnp.maximum(accb, rbuf[pl.ds(r * 32 + 16, 16)])

            def resolve(gx1, gy1, gx2, gy2, gs, ga, acc, extra):
                basev = ((gs > _SCORE_THRESH)
                         & jnp.logical_not(acc > _NMS_THRESH))
                if extra is not None:
                    ex1, ey1, ex2, ey2, ea, ekeep = extra
                masks = []
                for j in range(16):
                    xj1 = gx1[j]
                    yj1 = gy1[j]
                    xj2 = gx2[j]
                    yj2 = gy2[j]
                    aj = ga[j]
                    iou = _iou16(gx1, gy1, gx2, gy2, ga,
                                 xj1, yj1, xj2, yj2, aj, True)
                    masks.append((iou > _NMS_THRESH) & (iota < j))
                    if extra is not None:
                        iou2 = _iou16(ex1, ey1, ex2, ey2, ea,
                                      xj1, yj1, xj2, yj2, aj, True)
                        ebad = plsc.all_reduce_population_count(
                            (iou2 > _NMS_THRESH) & (ekeep > 0)) > 0
                        basev = basev & jnp.logical_not(ebad & (iota == j))
                keepb16 = jnp.zeros((16,), jnp.bool_)
                for j in range(16):
                    ncnt = plsc.all_reduce_population_count(
                        masks[j] & keepb16)
                    keepb16 = keepb16 | ((ncnt == 0) & basev & (iota == j))
                return keepb16.astype(jnp.int32)

            keepa = resolve(ax1, ay1, ax2, ay2, asc, aar, acca, None)
            keepb = resolve(bx1, by1, bx2, by2, bsc, bar, accb,
                            (ax1, ay1, ax2, ay2, aar, keepa))

            selw = iota == w
            mina = plsc.all_reduce_population_count(
                (keepa > 0) & selw)[0] > 0
            minb = plsc.all_reduce_population_count(
                (keepb > 0) & selw)[0] > 0

            @pl.when(mina)
            def _():
                ia = plsc.load_gather(
                    ov, [jnp.full((16,), o + w, jnp.int32)])
                r1 = plsc.load_gather(x1, [ia])
                r2 = plsc.load_gather(y1, [ia])
                r3 = plsc.load_gather(x2, [ia])
                r4 = plsc.load_gather(y2, [ia])
                ro = k_w * 16
                kb1[pl.ds(ro, 16)] = r1
                kb2[pl.ds(ro, 16)] = r2
                kb3[pl.ds(ro, 16)] = r3
                kb4[pl.ds(ro, 16)] = r4
                kba[pl.ds(ro, 16)] = (r3 - r1) * (r4 - r2)

            kw2 = k_w + mina.astype(jnp.int32)

            @pl.when(minb)
            def _():
                ib = plsc.load_gather(
                    ov, [jnp.full((16,), o + 16 + w, jnp.int32)])
                r1 = plsc.load_gather(x1, [ib])
                r2 = plsc.load_gather(y1, [ib])
                r3 = plsc.load_gather(x2, [ib])
                r4 = plsc.load_gather(y2, [ib])
                ro = kw2 * 16
                kb1[pl.ds(ro, 16)] = r1
                kb2[pl.ds(ro, 16)] = r2
                kb3[pl.ds(ro, 16)] = r3
                kb4[pl.ds(ro, 16)] = r4
                kba[pl.ds(ro, 16)] = (r3 - r1) * (r4 - r2)

            koff = kw2 + minb.astype(jnp.int32)

            nka = plsc.all_reduce_population_count(keepa > 0)[0]
            nkb = plsc.all_reduce_population_count(keepb > 0)[0]

            @pl.when(w == 0)
            def _():
                keep[pl.ds(o, 16)] = keepa
                keep[pl.ds(o + 16, 16)] = keepb

            return koff, ktot_g + nka + nkb

        _, ktot = lax.fori_loop(
            0, _NPAIR, pair_body, (jnp.int32(0), jnp.int32(0)))

        @pl.when(w == 0)
        def _():
            def sel_body(t, nk):
                o = t * 16
                kvec = keep[pl.ds(o, 16)]
                cum = jnp.cumsum(kvec)
                exc = cum - kvec
                gidx = o + iota
                kb = kvec > 0
                pos = jnp.where(kb, nk + exc, ktot + gidx - nk - exc)
                m = pos < _MAX_DET
                base = pos * 5
                idx16 = ov[pl.ds(o, 16)]
                vx1 = plsc.load_gather(x1, [idx16])
                vy1 = plsc.load_gather(y1, [idx16])
                vx2 = plsc.load_gather(x2, [idx16])
                vy2 = plsc.load_gather(y2, [idx16])
                vs = plsc.load_gather(sv, [idx16])
                so = jnp.where(kb, vs, jnp.float32(_NEG))
                plsc.store_scatter(outbuf, [base], vx1, mask=m)
                plsc.store_scatter(outbuf, [base + 1], vy1, mask=m)
                plsc.store_scatter(outbuf, [base + 2], vx2, mask=m)
                plsc.store_scatter(outbuf, [base + 3], vy2, mask=m)
                plsc.store_scatter(outbuf, [base + 4], so, mask=m)
                return nk + jnp.sum(kvec)

            lax.fori_loop(0, _NBLK, sel_body, jnp.int32(0))
            pltpu.sync_copy(outbuf, outh)


_nms_call = functools.partial(
    pl.kernel,
    out_type=jax.ShapeDtypeStruct((512,), jnp.float32),
    mesh=plsc.VectorSubcoreMesh(core_axis_name="c", subcore_axis_name="s"),
    compiler_params=pltpu.CompilerParams(needs_layout_passes=False),
    scratch_types=[
        pltpu.VMEM((_NPAD,), jnp.float32),
        pltpu.VMEM((_NPAD,), jnp.float32),
        pltpu.VMEM((_NPAD,), jnp.float32),
        pltpu.VMEM((_NPAD,), jnp.float32),
        pltpu.VMEM((_NPAD,), jnp.float32),
        pltpu.VMEM((_NPAD,), jnp.int32),
        pltpu.VMEM((_KCAP * 16,), jnp.float32),
        pltpu.VMEM((_KCAP * 16,), jnp.float32),
        pltpu.VMEM((_KCAP * 16,), jnp.float32),
        pltpu.VMEM((_KCAP * 16,), jnp.float32),
        pltpu.VMEM((_KCAP * 16,), jnp.float32),
        pltpu.VMEM((_NPAD,), jnp.int32),
        pltpu.VMEM((512,), jnp.float32),
        pltpu.VMEM((32,), jnp.float32),
        pltpu.VMEM((512,), jnp.float32),
        pltpu.VMEM_SHARED((1024,), jnp.float32),
    ],
)(_nms_body)


def kernel(boxes, scores):
    order = jnp.argsort(-scores).astype(jnp.int32)
    pad = _NPAD - _N
    orderp = jnp.concatenate(
        [order, jnp.arange(_N, _NPAD, dtype=jnp.int32)])
    bp = jnp.concatenate([boxes, jnp.zeros((pad, 4), jnp.float32)], axis=0)
    sp = jnp.concatenate(
        [scores, jnp.full((pad,), -1.0, jnp.float32)])
    out = _nms_call(bp[:, 0], bp[:, 1], bp[:, 2], bp[:, 3], sp, orderp)
    return out[:_MAX_DET * 5].reshape(_MAX_DET, 5)
